# baseline (speedup 1.0000x reference)
import functools

import jax
import jax.numpy as jnp
from jax import lax
from jax.experimental import pallas as pl
from jax.experimental.pallas import tpu as pltpu
from jax.experimental.pallas import tpu_sc as plsc

D = 128
B = 4096 * 200
NC, NS = 2, 16
NW = NC * NS
B_PER_W = B // NW
CHUNK = 200
N_CHUNKS = B_PER_W // CHUNK
NBUF = 4


def _make_gather():
  mesh = plsc.VectorSubcoreMesh(core_axis_name="c", subcore_axis_name="s")

  @functools.partial(
      pl.kernel,
      mesh=mesh,
      out_type=jax.ShapeDtypeStruct((B, D), jnp.float32),
      scratch_types=[
          pltpu.VMEM((B_PER_W,), jnp.int32),
          pltpu.VMEM((NBUF, CHUNK, D), jnp.float32),
      ] + [pltpu.SemaphoreType.DMA] * (2 * NBUF),
  )
  def gather_kernel(table_hbm, idx_hbm, out_hbm, idx_v, rows_v, *sems):
    gsems = sems[:NBUF]
    ssems = sems[NBUF:]
    wid = lax.axis_index("s") * NC + lax.axis_index("c")
    base = wid * B_PER_W

    pltpu.sync_copy(idx_hbm.at[pl.ds(base, B_PER_W)], idx_v)

    def start_gather(c, j):
      pltpu.async_copy(
          table_hbm.at[idx_v.at[pl.ds(c * CHUNK, CHUNK)]],
          rows_v.at[j], gsems[j])

    for j in range(NBUF):
      start_gather(j, j)

    def body(g, carry):
      for j in range(NBUF):
        c = g * NBUF + j
        off = base + c * CHUNK
        pltpu.make_async_copy(
            table_hbm.at[idx_v.at[pl.ds(0, CHUNK)]],
            rows_v.at[j], gsems[j]).wait()
        out_slice = out_hbm.at[pl.ds(off, CHUNK)]
        pltpu.async_copy(rows_v.at[j], out_slice, ssems[j])
        pltpu.make_async_copy(rows_v.at[j], out_slice, ssems[j]).wait()
        @pl.when(c + NBUF < N_CHUNKS)
        def _():
          start_gather(c + NBUF, j)
      return carry

    lax.fori_loop(0, N_CHUNKS // NBUF, body, 0)

  return gather_kernel


_gather = _make_gather()


def kernel(idx, table):
  idx_flat = idx.reshape(B).astype(jnp.int32)
  out = _gather(table, idx_flat)
  return out.reshape(idx.shape + (D,))


# speedup vs baseline: 1.0058x; 1.0058x over previous
import functools

import jax
import jax.numpy as jnp
from jax import lax
from jax.experimental import pallas as pl
from jax.experimental.pallas import tpu as pltpu
from jax.experimental.pallas import tpu_sc as plsc

D = 128
B = 4096 * 200
NC, NS = 2, 16
NW = NC * NS
B_PER_W = B // NW
CHUNK = 200
N_CHUNKS = B_PER_W // CHUNK
NBUF = 4


def _make_gather():
  mesh = plsc.VectorSubcoreMesh(core_axis_name="c", subcore_axis_name="s")

  @functools.partial(
      pl.kernel,
      mesh=mesh,
      out_type=jax.ShapeDtypeStruct((B, D), jnp.float32),
      scratch_types=[
          pltpu.VMEM((B_PER_W,), jnp.int32),
          pltpu.VMEM((NBUF, CHUNK, D), jnp.float32),
      ] + [pltpu.SemaphoreType.DMA] * (2 * NBUF),
  )
  def gather_kernel(table_hbm, idx_hbm, out_hbm, idx_v, rows_v, *sems):
    gsems = sems[:NBUF]
    ssems = sems[NBUF:]
    wid = lax.axis_index("s") * NC + lax.axis_index("c")
    base = wid * B_PER_W

    pltpu.sync_copy(idx_hbm.at[pl.ds(base, B_PER_W)], idx_v)

    def start_gather(c, j):
      pltpu.async_copy(
          table_hbm.at[idx_v.at[pl.ds(c * CHUNK, CHUNK)]],
          rows_v.at[j], gsems[j])

    def wait_store(j):
      pltpu.make_async_copy(
          rows_v.at[j], out_hbm.at[pl.ds(base, CHUNK)], ssems[j]).wait()

    for j in range(NBUF - 1):
      start_gather(j, j)

    def body(g, carry):
      for j in range(NBUF):
        c = g * NBUF + j
        pltpu.make_async_copy(
            table_hbm.at[idx_v.at[pl.ds(0, CHUNK)]],
            rows_v.at[j], gsems[j]).wait()
        pltpu.async_copy(rows_v.at[j],
                         out_hbm.at[pl.ds(base + c * CHUNK, CHUNK)],
                         ssems[j])
        cr = c + NBUF - 1
        jr = (j + NBUF - 1) % NBUF
        @pl.when(cr < N_CHUNKS)
        def _():
          @pl.when(cr >= NBUF)
          def _():
            wait_store(jr)
          start_gather(cr, jr)
      return carry

    lax.fori_loop(0, N_CHUNKS // NBUF, body, 0)
    for j in range(NBUF):
      wait_store(j)

  return gather_kernel


_gather = _make_gather()


def kernel(idx, table):
  idx_flat = idx.reshape(B).astype(jnp.int32)
  out = _gather(table, idx_flat)
  return out.reshape(idx.shape + (D,))


# speedup vs baseline: 1.6355x; 1.6261x over previous
import functools

import jax
import jax.numpy as jnp
from jax import lax
from jax.experimental import pallas as pl
from jax.experimental.pallas import tpu as pltpu
from jax.experimental.pallas import tpu_sc as plsc

D = 128
B = 4096 * 200
NC, NS = 2, 16
NW = NC * NS
B_PER_W = B // NW
CHUNK = 128
N_CHUNKS = B_PER_W // CHUNK
NBUF = 2


def _make_gather():
  mesh = plsc.VectorSubcoreMesh(core_axis_name="c", subcore_axis_name="s")

  @functools.partial(
      pl.kernel,
      mesh=mesh,
      out_type=jax.ShapeDtypeStruct((B, D), jnp.float32),
      scratch_types=[
          pltpu.VMEM((B_PER_W,), jnp.int32),
          pltpu.VMEM((NBUF, CHUNK, D), jnp.float32),
          pltpu.VMEM_SHARED((8192, D), jnp.float32),
      ] + [pltpu.SemaphoreType.DMA] * (2 * NBUF),
  )
  def gather_kernel(table_hbm, idx_hbm, out_hbm, idx_v, rows_v, table_sh,
                    *sems):
    gsems = sems[:NBUF]
    ssems = sems[NBUF:]
    sid = lax.axis_index("s")
    wid = sid * NC + lax.axis_index("c")
    base = wid * B_PER_W

    trows = 8192 // NS
    pltpu.sync_copy(table_hbm.at[pl.ds(sid * trows, trows)],
                    table_sh.at[pl.ds(sid * trows, trows)])
    pltpu.sync_copy(idx_hbm.at[pl.ds(base, B_PER_W)], idx_v)
    plsc.subcore_barrier()

    def start_gather(c, j):
      pltpu.async_copy(
          table_sh.at[idx_v.at[pl.ds(c * CHUNK, CHUNK)]],
          rows_v.at[j], gsems[j])

    def wait_store(j):
      pltpu.make_async_copy(
          rows_v.at[j], out_hbm.at[pl.ds(base, CHUNK)], ssems[j]).wait()

    for j in range(NBUF - 1):
      start_gather(j, j)

    def body(g, carry):
      for j in range(NBUF):
        c = g * NBUF + j
        pltpu.make_async_copy(
            table_sh.at[idx_v.at[pl.ds(0, CHUNK)]],
            rows_v.at[j], gsems[j]).wait()
        pltpu.async_copy(rows_v.at[j],
                         out_hbm.at[pl.ds(base + c * CHUNK, CHUNK)],
                         ssems[j])
        cr = c + NBUF - 1
        jr = (j + NBUF - 1) % NBUF
        @pl.when(cr < N_CHUNKS)
        def _():
          @pl.when(cr >= NBUF)
          def _():
            wait_store(jr)
          start_gather(cr, jr)
      return carry

    lax.fori_loop(0, N_CHUNKS // NBUF, body, 0)
    for j in range(NBUF):
      wait_store(j)

  return gather_kernel


_gather = _make_gather()


def kernel(idx, table):
  idx_flat = idx.reshape(B).astype(jnp.int32)
  out = _gather(table, idx_flat)
  return out.reshape(idx.shape + (D,))


# speedup vs baseline: 1.6448x; 1.0057x over previous
import functools

import jax
import jax.numpy as jnp
from jax import lax
from jax.experimental import pallas as pl
from jax.experimental.pallas import tpu as pltpu
from jax.experimental.pallas import tpu_sc as plsc

D = 128
B = 4096 * 200
NC, NS = 2, 16
NW = NC * NS
B_PER_W = B // NW
CHUNK = 128
N_CHUNKS = B_PER_W // CHUNK
NBUF = 2


def _make_gather():
  mesh = plsc.VectorSubcoreMesh(core_axis_name="c", subcore_axis_name="s")

  @functools.partial(
      pl.kernel,
      mesh=mesh,
      out_type=jax.ShapeDtypeStruct((B, D), jnp.float32),
      scratch_types=[
          pltpu.VMEM((B_PER_W,), jnp.int32),
          pltpu.VMEM((NBUF, CHUNK, D), jnp.float32),
          pltpu.VMEM_SHARED((8192, D), jnp.float32),
      ] + [pltpu.SemaphoreType.DMA] * (2 * NBUF),
  )
  def gather_kernel(table_hbm, idx_hbm, out_hbm, idx_v, rows_v, table_sh,
                    *sems):
    gsems = sems[:NBUF]
    ssems = sems[NBUF:]
    sid = lax.axis_index("s")
    wid = sid * NC + lax.axis_index("c")
    base = wid * B_PER_W

    trows = 8192 // NS
    t_src = table_hbm.at[pl.ds(sid * trows, trows)]
    t_dst = table_sh.at[pl.ds(sid * trows, trows)]
    i_src = idx_hbm.at[pl.ds(base, B_PER_W)]
    pltpu.async_copy(t_src, t_dst, gsems[0])
    pltpu.async_copy(i_src, idx_v, ssems[0])
    pltpu.make_async_copy(t_src, t_dst, gsems[0]).wait()
    pltpu.make_async_copy(i_src, idx_v, ssems[0]).wait()
    plsc.subcore_barrier()

    def start_gather(c, j):
      pltpu.async_copy(
          table_sh.at[idx_v.at[pl.ds(c * CHUNK, CHUNK)]],
          rows_v.at[j], gsems[j])

    def wait_store(j):
      pltpu.make_async_copy(
          rows_v.at[j], out_hbm.at[pl.ds(base, CHUNK)], ssems[j]).wait()

    for j in range(NBUF - 1):
      start_gather(j, j)

    def body(g, carry):
      for j in range(NBUF):
        c = g * NBUF + j
        pltpu.make_async_copy(
            table_sh.at[idx_v.at[pl.ds(0, CHUNK)]],
            rows_v.at[j], gsems[j]).wait()
        pltpu.async_copy(rows_v.at[j],
                         out_hbm.at[pl.ds(base + c * CHUNK, CHUNK)],
                         ssems[j])
        cr = c + NBUF - 1
        jr = (j + NBUF - 1) % NBUF
        @pl.when(cr < N_CHUNKS)
        def _():
          @pl.when(cr >= NBUF)
          def _():
            wait_store(jr)
          start_gather(cr, jr)
      return carry

    lax.fori_loop(0, N_CHUNKS // NBUF, body, 0)
    for j in range(NBUF):
      wait_store(j)

  return gather_kernel


_gather = _make_gather()


def kernel(idx, table):
  idx_flat = idx.reshape(B).astype(jnp.int32)
  out = _gather(table, idx_flat)
  return out.reshape(idx.shape + (D,))


# speedup vs baseline: 1.7847x; 1.0850x over previous
import functools

import jax
import jax.numpy as jnp
from jax import lax
from jax.experimental import pallas as pl
from jax.experimental.pallas import tpu as pltpu
from jax.experimental.pallas import tpu_sc as plsc

D = 128
B = 4096 * 200
NC, NS = 2, 16
NW = NC * NS
B_PER_W = B // NW
CHUNK = 64
N_CHUNKS = B_PER_W // CHUNK
NBUF = 4


def _make_gather():
  mesh = plsc.VectorSubcoreMesh(core_axis_name="c", subcore_axis_name="s")

  @functools.partial(
      pl.kernel,
      mesh=mesh,
      out_type=jax.ShapeDtypeStruct((B, D), jnp.float32),
      scratch_types=[
          pltpu.VMEM((B_PER_W,), jnp.int32),
          pltpu.VMEM((NBUF, CHUNK, D), jnp.float32),
          pltpu.VMEM_SHARED((8192, D), jnp.float32),
      ] + [pltpu.SemaphoreType.DMA] * (2 * NBUF),
  )
  def gather_kernel(table_hbm, idx_hbm, out_hbm, idx_v, rows_v, table_sh,
                    *sems):
    gsems = sems[:NBUF]
    ssems = sems[NBUF:]
    sid = lax.axis_index("s")
    wid = sid * NC + lax.axis_index("c")
    base = wid * B_PER_W

    trows = 8192 // NS
    t_src = table_hbm.at[pl.ds(sid * trows, trows)]
    t_dst = table_sh.at[pl.ds(sid * trows, trows)]
    i_src = idx_hbm.at[pl.ds(base, B_PER_W)]
    pltpu.async_copy(t_src, t_dst, gsems[0])
    pltpu.async_copy(i_src, idx_v, ssems[0])
    pltpu.make_async_copy(t_src, t_dst, gsems[0]).wait()
    pltpu.make_async_copy(i_src, idx_v, ssems[0]).wait()
    plsc.subcore_barrier()

    def start_gather(c, j):
      pltpu.async_copy(
          table_sh.at[idx_v.at[pl.ds(c * CHUNK, CHUNK)]],
          rows_v.at[j], gsems[j])

    def wait_store(j):
      pltpu.make_async_copy(
          rows_v.at[j], out_hbm.at[pl.ds(base, CHUNK)], ssems[j]).wait()

    for j in range(NBUF - 1):
      start_gather(j, j)

    def body(g, carry):
      for j in range(NBUF):
        c = g * NBUF + j
        pltpu.make_async_copy(
            table_sh.at[idx_v.at[pl.ds(0, CHUNK)]],
            rows_v.at[j], gsems[j]).wait()
        pltpu.async_copy(rows_v.at[j],
                         out_hbm.at[pl.ds(base + c * CHUNK, CHUNK)],
                         ssems[j])
        cr = c + NBUF - 1
        jr = (j + NBUF - 1) % NBUF
        @pl.when(cr < N_CHUNKS)
        def _():
          @pl.when(cr >= NBUF)
          def _():
            wait_store(jr)
          start_gather(cr, jr)
      return carry

    lax.fori_loop(0, N_CHUNKS // NBUF, body, 0)
    for j in range(NBUF):
      wait_store(j)

  return gather_kernel


_gather = _make_gather()


def kernel(idx, table):
  idx_flat = idx.reshape(B).astype(jnp.int32)
  out = _gather(table, idx_flat)
  return out.reshape(idx.shape + (D,))


# speedup vs baseline: 1.7872x; 1.0014x over previous
import functools

import jax
import jax.numpy as jnp
from jax import lax
from jax.experimental import pallas as pl
from jax.experimental.pallas import tpu as pltpu
from jax.experimental.pallas import tpu_sc as plsc

D = 128
B = 4096 * 200
NC, NS = 2, 16
NW = NC * NS
B_PER_W = B // NW
CHUNK = 32
N_CHUNKS = B_PER_W // CHUNK
NBUF = 8


def _make_gather():
  mesh = plsc.VectorSubcoreMesh(core_axis_name="c", subcore_axis_name="s")

  @functools.partial(
      pl.kernel,
      mesh=mesh,
      out_type=jax.ShapeDtypeStruct((B, D), jnp.float32),
      scratch_types=[
          pltpu.VMEM((B_PER_W,), jnp.int32),
          pltpu.VMEM((NBUF, CHUNK, D), jnp.float32),
          pltpu.VMEM_SHARED((8192, D), jnp.float32),
      ] + [pltpu.SemaphoreType.DMA] * (2 * NBUF),
  )
  def gather_kernel(table_hbm, idx_hbm, out_hbm, idx_v, rows_v, table_sh,
                    *sems):
    gsems = sems[:NBUF]
    ssems = sems[NBUF:]
    sid = lax.axis_index("s")
    wid = sid * NC + lax.axis_index("c")
    base = wid * B_PER_W

    trows = 8192 // NS
    t_src = table_hbm.at[pl.ds(sid * trows, trows)]
    t_dst = table_sh.at[pl.ds(sid * trows, trows)]
    i_src = idx_hbm.at[pl.ds(base, B_PER_W)]
    pltpu.async_copy(t_src, t_dst, gsems[0])
    pltpu.async_copy(i_src, idx_v, ssems[0])
    pltpu.make_async_copy(t_src, t_dst, gsems[0]).wait()
    pltpu.make_async_copy(i_src, idx_v, ssems[0]).wait()
    plsc.subcore_barrier()

    def start_gather(c, j):
      pltpu.async_copy(
          table_sh.at[idx_v.at[pl.ds(c * CHUNK, CHUNK)]],
          rows_v.at[j], gsems[j])

    def wait_store(j):
      pltpu.make_async_copy(
          rows_v.at[j], out_hbm.at[pl.ds(base, CHUNK)], ssems[j]).wait()

    for j in range(NBUF - 1):
      start_gather(j, j)

    def body(g, carry):
      for j in range(NBUF):
        c = g * NBUF + j
        pltpu.make_async_copy(
            table_sh.at[idx_v.at[pl.ds(0, CHUNK)]],
            rows_v.at[j], gsems[j]).wait()
        pltpu.async_copy(rows_v.at[j],
                         out_hbm.at[pl.ds(base + c * CHUNK, CHUNK)],
                         ssems[j])
        cr = c + NBUF - 1
        jr = (j + NBUF - 1) % NBUF
        @pl.when(cr < N_CHUNKS)
        def _():
          @pl.when(cr >= NBUF)
          def _():
            wait_store(jr)
          start_gather(cr, jr)
      return carry

    lax.fori_loop(0, N_CHUNKS // NBUF, body, 0)
    for j in range(NBUF):
      wait_store(j)

  return gather_kernel


_gather = _make_gather()


def kernel(idx, table):
  idx_flat = idx.reshape(B).astype(jnp.int32)
  out = _gather(table, idx_flat)
  return out.reshape(idx.shape + (D,))


# speedup vs baseline: 1.7927x; 1.0031x over previous
import functools

import jax
import jax.numpy as jnp
from jax import lax
from jax.experimental import pallas as pl
from jax.experimental.pallas import tpu as pltpu
from jax.experimental.pallas import tpu_sc as plsc

D = 128
B = 4096 * 200
NC, NS = 2, 16
NW = NC * NS
B_PER_W = B // NW
CHUNK = 16
N_CHUNKS = B_PER_W // CHUNK
NBUF = 8


def _make_gather():
  mesh = plsc.VectorSubcoreMesh(core_axis_name="c", subcore_axis_name="s")

  @functools.partial(
      pl.kernel,
      mesh=mesh,
      out_type=jax.ShapeDtypeStruct((B, D), jnp.float32),
      scratch_types=[
          pltpu.VMEM((B_PER_W,), jnp.int32),
          pltpu.VMEM((NBUF, CHUNK, D), jnp.float32),
          pltpu.VMEM_SHARED((8192, D), jnp.float32),
      ] + [pltpu.SemaphoreType.DMA] * (2 * NBUF),
  )
  def gather_kernel(table_hbm, idx_hbm, out_hbm, idx_v, rows_v, table_sh,
                    *sems):
    gsems = sems[:NBUF]
    ssems = sems[NBUF:]
    sid = lax.axis_index("s")
    wid = sid * NC + lax.axis_index("c")
    base = wid * B_PER_W

    trows = 8192 // NS
    t_src = table_hbm.at[pl.ds(sid * trows, trows)]
    t_dst = table_sh.at[pl.ds(sid * trows, trows)]
    i_src = idx_hbm.at[pl.ds(base, B_PER_W)]
    pltpu.async_copy(t_src, t_dst, gsems[0])
    pltpu.async_copy(i_src, idx_v, ssems[0])
    pltpu.make_async_copy(t_src, t_dst, gsems[0]).wait()
    pltpu.make_async_copy(i_src, idx_v, ssems[0]).wait()
    plsc.subcore_barrier()

    def start_gather(c, j):
      pltpu.async_copy(
          table_sh.at[idx_v.at[pl.ds(c * CHUNK, CHUNK)]],
          rows_v.at[j], gsems[j])

    def wait_store(j):
      pltpu.make_async_copy(
          rows_v.at[j], out_hbm.at[pl.ds(base, CHUNK)], ssems[j]).wait()

    for j in range(NBUF - 1):
      start_gather(j, j)

    def body(g, carry):
      for j in range(NBUF):
        c = g * NBUF + j
        pltpu.make_async_copy(
            table_sh.at[idx_v.at[pl.ds(0, CHUNK)]],
            rows_v.at[j], gsems[j]).wait()
        pltpu.async_copy(rows_v.at[j],
                         out_hbm.at[pl.ds(base + c * CHUNK, CHUNK)],
                         ssems[j])
        cr = c + NBUF - 1
        jr = (j + NBUF - 1) % NBUF
        @pl.when(cr < N_CHUNKS)
        def _():
          @pl.when(cr >= NBUF)
          def _():
            wait_store(jr)
          start_gather(cr, jr)
      return carry

    lax.fori_loop(0, N_CHUNKS // NBUF, body, 0)
    for j in range(NBUF):
      wait_store(j)

  return gather_kernel


_gather = _make_gather()


def kernel(idx, table):
  idx_flat = idx.reshape(B).astype(jnp.int32)
  out = _gather(table, idx_flat)
  return out.reshape(idx.shape + (D,))
